# baseline (speedup 1.0000x reference)
import jax
import jax.numpy as jnp
from jax.experimental import pallas as pl
from jax.experimental.pallas import tpu as pltpu

NUM_CLASSES = 64
HIDDEN = 2048
VOCAB = 100000
BATCH = 1024

BLOCK_V = 1024
NUM_FULL = VOCAB // BLOCK_V
REM = VOCAB - NUM_FULL * BLOCK_V
LANE = 128
NUM_SRC = 2
NUM_SEM = 8


def _probe_scatter_kernel(hidden_ref, w_ref, vid_ref, vidv_ref, out_ref,
                          logits_ref, neg_ref, rem_ref, tiles_ref,
                          sem_fill, sem_col):
    for s in range(NUM_SRC):
        neg_ref[s] = jnp.full((BATCH, BLOCK_V), -jnp.inf, jnp.float32)
    rem_ref[...] = jnp.full((BATCH, REM), -jnp.inf, jnp.float32)

    fills = [
        pltpu.make_async_copy(
            neg_ref.at[j % NUM_SRC],
            out_ref.at[:, pl.ds(j * BLOCK_V, BLOCK_V)],
            sem_fill.at[j % NUM_SEM])
        for j in range(NUM_FULL)
    ]
    fills.append(pltpu.make_async_copy(
        rem_ref, out_ref.at[:, pl.ds(NUM_FULL * BLOCK_V, REM)],
        sem_fill.at[NUM_FULL % NUM_SEM]))
    for f in fills:
        f.start()

    logits_ref[...] = jax.lax.dot_general(
        hidden_ref[...], w_ref[...],
        dimension_numbers=(((1,), (1,)), ((), ())),
        preferred_element_type=jnp.float32,
    )
    vids = vidv_ref[...]
    for k in range(NUM_CLASSES):
        base_k = (vid_ref[0, k] // LANE) * LANE
        cols = base_k + jax.lax.broadcasted_iota(jnp.int32, (1, LANE), 1)
        hits = cols == vids
        scattered = jax.lax.dot_general(
            logits_ref[...], hits.astype(jnp.float32),
            dimension_numbers=(((1,), (0,)), ((), ())),
            preferred_element_type=jnp.float32,
        )
        covered = jnp.any(hits, axis=0, keepdims=True)
        tiles_ref[k] = jnp.where(covered, scattered, -jnp.inf)

    for f in fills:
        f.wait()

    cols_dmas = []
    for k in range(NUM_CLASSES):
        base_k = pl.multiple_of((vid_ref[0, k] // LANE) * LANE, LANE)
        cols_dmas.append(pltpu.make_async_copy(
            tiles_ref.at[k], out_ref.at[:, pl.ds(base_k, LANE)], sem_col))
    for c in cols_dmas:
        c.start()
    for c in cols_dmas:
        c.wait()


@jax.jit
def kernel(hidden_states, probe_weights, vocab_ids):
    h = hidden_states.astype(jnp.float32)
    vid = vocab_ids.astype(jnp.int32).reshape(1, NUM_CLASSES)
    vidv = vocab_ids.astype(jnp.int32).reshape(NUM_CLASSES, 1)
    return pl.pallas_call(
        _probe_scatter_kernel,
        grid=(1,),
        in_specs=[
            pl.BlockSpec((BATCH, HIDDEN), lambda i: (0, 0)),
            pl.BlockSpec((NUM_CLASSES, HIDDEN), lambda i: (0, 0)),
            pl.BlockSpec(memory_space=pltpu.SMEM),
            pl.BlockSpec((NUM_CLASSES, 1), lambda i: (0, 0)),
        ],
        out_specs=pl.BlockSpec(memory_space=pl.ANY),
        out_shape=jax.ShapeDtypeStruct((BATCH, VOCAB), jnp.float32),
        scratch_shapes=[
            pltpu.VMEM((BATCH, NUM_CLASSES), jnp.float32),
            pltpu.VMEM((NUM_SRC, BATCH, BLOCK_V), jnp.float32),
            pltpu.VMEM((BATCH, REM), jnp.float32),
            pltpu.VMEM((NUM_CLASSES, BATCH, LANE), jnp.float32),
            pltpu.SemaphoreType.DMA((NUM_SEM,)),
            pltpu.SemaphoreType.DMA,
        ],
        compiler_params=pltpu.CompilerParams(
            dimension_semantics=("arbitrary",),
        ),
    )(h, probe_weights, vid, vidv)


# speedup vs baseline: 1.0060x; 1.0060x over previous
import jax
import jax.numpy as jnp
from jax.experimental import pallas as pl
from jax.experimental.pallas import tpu as pltpu

NUM_CLASSES = 64
HIDDEN = 2048
VOCAB = 100000
BATCH = 1024

FILL_ROWS = 16
LANE = 128
NUM_SEM = 8


def _probe_scatter_kernel(hidden_ref, w_ref, vid_ref, vidv_ref, out_ref,
                          logits_ref, neg_ref, tiles_ref,
                          sem_fill, sem_col):
    neg_ref[...] = jnp.full((FILL_ROWS, VOCAB), -jnp.inf, jnp.float32)

    fills = [
        pltpu.make_async_copy(
            neg_ref,
            out_ref.at[pl.ds(i * FILL_ROWS, FILL_ROWS), :],
            sem_fill.at[i % NUM_SEM])
        for i in range(BATCH // FILL_ROWS)
    ]
    for f in fills:
        f.start()

    logits_ref[...] = jax.lax.dot_general(
        hidden_ref[...], w_ref[...],
        dimension_numbers=(((1,), (1,)), ((), ())),
        preferred_element_type=jnp.float32,
    )
    vids = vidv_ref[...]
    for k in range(NUM_CLASSES):
        base_k = (vid_ref[0, k] // LANE) * LANE
        cols = base_k + jax.lax.broadcasted_iota(jnp.int32, (1, LANE), 1)
        hits = cols == vids
        scattered = jax.lax.dot_general(
            logits_ref[...], hits.astype(jnp.float32),
            dimension_numbers=(((1,), (0,)), ((), ())),
            preferred_element_type=jnp.float32,
        )
        covered = jnp.any(hits, axis=0, keepdims=True)
        tiles_ref[k] = jnp.where(covered, scattered, -jnp.inf)

    for f in fills:
        f.wait()

    cols_dmas = []
    for k in range(NUM_CLASSES):
        base_k = pl.multiple_of((vid_ref[0, k] // LANE) * LANE, LANE)
        cols_dmas.append(pltpu.make_async_copy(
            tiles_ref.at[k], out_ref.at[:, pl.ds(base_k, LANE)], sem_col))
    for c in cols_dmas:
        c.start()
    for c in cols_dmas:
        c.wait()


@jax.jit
def kernel(hidden_states, probe_weights, vocab_ids):
    h = hidden_states.astype(jnp.float32)
    vid = vocab_ids.astype(jnp.int32).reshape(1, NUM_CLASSES)
    vidv = vocab_ids.astype(jnp.int32).reshape(NUM_CLASSES, 1)
    return pl.pallas_call(
        _probe_scatter_kernel,
        grid=(1,),
        in_specs=[
            pl.BlockSpec((BATCH, HIDDEN), lambda i: (0, 0)),
            pl.BlockSpec((NUM_CLASSES, HIDDEN), lambda i: (0, 0)),
            pl.BlockSpec(memory_space=pltpu.SMEM),
            pl.BlockSpec((NUM_CLASSES, 1), lambda i: (0, 0)),
        ],
        out_specs=pl.BlockSpec(memory_space=pl.ANY),
        out_shape=jax.ShapeDtypeStruct((BATCH, VOCAB), jnp.float32),
        scratch_shapes=[
            pltpu.VMEM((BATCH, NUM_CLASSES), jnp.float32),
            pltpu.VMEM((FILL_ROWS, VOCAB), jnp.float32),
            pltpu.VMEM((NUM_CLASSES, BATCH, LANE), jnp.float32),
            pltpu.SemaphoreType.DMA((NUM_SEM,)),
            pltpu.SemaphoreType.DMA,
        ],
        compiler_params=pltpu.CompilerParams(
            dimension_semantics=("arbitrary",),
        ),
    )(h, probe_weights, vid, vidv)


# speedup vs baseline: 1.0091x; 1.0031x over previous
import jax
import jax.numpy as jnp
from jax.experimental import pallas as pl
from jax.experimental.pallas import tpu as pltpu

NUM_CLASSES = 64
HIDDEN = 2048
VOCAB = 100000
BATCH = 1024

FILL_ROWS = 16
LANE = 128
NUM_SEM = 8
NUM_THREADS = 2


def _probe_scatter_kernel(hidden_ref, w_ref, vid_ref, vidv_ref, out_ref,
                          logits_ref, neg_ref, tiles_ref,
                          sem_fill, sem_col):
    neg_ref[...] = jnp.full((FILL_ROWS, VOCAB), -jnp.inf, jnp.float32)

    fills = [
        pltpu.make_async_copy(
            neg_ref,
            out_ref.at[pl.ds(i * FILL_ROWS, FILL_ROWS), :],
            sem_fill.at[i % NUM_SEM])
        for i in range(BATCH // FILL_ROWS)
    ]
    for i, f in enumerate(fills):
        f.start(priority=i % NUM_THREADS)

    logits_ref[...] = jax.lax.dot_general(
        hidden_ref[...], w_ref[...],
        dimension_numbers=(((1,), (1,)), ((), ())),
        preferred_element_type=jnp.float32,
    )
    vids = vidv_ref[...]
    for k in range(NUM_CLASSES):
        base_k = (vid_ref[0, k] // LANE) * LANE
        cols = base_k + jax.lax.broadcasted_iota(jnp.int32, (1, LANE), 1)
        hits = cols == vids
        scattered = jax.lax.dot_general(
            logits_ref[...], hits.astype(jnp.float32),
            dimension_numbers=(((1,), (0,)), ((), ())),
            preferred_element_type=jnp.float32,
        )
        covered = jnp.any(hits, axis=0, keepdims=True)
        tiles_ref[k] = jnp.where(covered, scattered, -jnp.inf)

    for f in fills:
        f.wait()

    cols_dmas = []
    for k in range(NUM_CLASSES):
        base_k = pl.multiple_of((vid_ref[0, k] // LANE) * LANE, LANE)
        cols_dmas.append(pltpu.make_async_copy(
            tiles_ref.at[k], out_ref.at[:, pl.ds(base_k, LANE)], sem_col))
    for k, c in enumerate(cols_dmas):
        c.start(priority=k % NUM_THREADS)
    for c in cols_dmas:
        c.wait()


@jax.jit
def kernel(hidden_states, probe_weights, vocab_ids):
    h = hidden_states.astype(jnp.float32)
    vid = vocab_ids.astype(jnp.int32).reshape(1, NUM_CLASSES)
    vidv = vocab_ids.astype(jnp.int32).reshape(NUM_CLASSES, 1)
    return pl.pallas_call(
        _probe_scatter_kernel,
        grid=(1,),
        in_specs=[
            pl.BlockSpec((BATCH, HIDDEN), lambda i: (0, 0)),
            pl.BlockSpec((NUM_CLASSES, HIDDEN), lambda i: (0, 0)),
            pl.BlockSpec(memory_space=pltpu.SMEM),
            pl.BlockSpec((NUM_CLASSES, 1), lambda i: (0, 0)),
        ],
        out_specs=pl.BlockSpec(memory_space=pl.ANY),
        out_shape=jax.ShapeDtypeStruct((BATCH, VOCAB), jnp.float32),
        scratch_shapes=[
            pltpu.VMEM((BATCH, NUM_CLASSES), jnp.float32),
            pltpu.VMEM((FILL_ROWS, VOCAB), jnp.float32),
            pltpu.VMEM((NUM_CLASSES, BATCH, LANE), jnp.float32),
            pltpu.SemaphoreType.DMA((NUM_SEM,)),
            pltpu.SemaphoreType.DMA,
        ],
        compiler_params=pltpu.CompilerParams(
            dimension_semantics=("arbitrary",),
        ),
    )(h, probe_weights, vid, vidv)
